# baseline (speedup 1.0000x reference)
import functools

import jax
import jax.numpy as jnp
from jax import lax
from jax.experimental import pallas as pl
from jax.experimental.pallas import tpu as pltpu
from jax.experimental.pallas import tpu_sc as plsc

B = 16384
EMB_DIM = 128
PHYLO_DIM = 64
FUSED_IN = 320
NUM_PHYLO = 100

_NC = 2
_NS = 16
_NW = _NC * _NS
_BPW = B // _NW
_CH = 128

_CHUNK = 1024
_NCHUNK = B // _CHUNK
_NBUF = 4


def _sc_gather_body(species_hbm, sid_hbm, s_out, sidx_v, srows_v, sem):
    wid = lax.axis_index("s") * _NC + lax.axis_index("c")
    base = wid * _BPW
    pltpu.sync_copy(sid_hbm.at[pl.ds(base, _BPW)], sidx_v)
    gathers = []
    for j in range(_BPW // _CH):
        gathers.append(pltpu.async_copy(
            species_hbm.at[sidx_v.at[pl.ds(j * _CH, _CH)]],
            srows_v.at[pl.ds(j * _CH, _CH)], sem))
    for g in gathers:
        g.wait()
    pltpu.sync_copy(srows_v, s_out.at[pl.ds(base, _BPW)])


@functools.lru_cache(maxsize=None)
def _get_sc_gather():
    return pl.kernel(
        _sc_gather_body,
        out_type=jax.ShapeDtypeStruct((B, EMB_DIM), jnp.float32),
        mesh=plsc.VectorSubcoreMesh(core_axis_name="c", subcore_axis_name="s"),
        scratch_types=[
            pltpu.VMEM((_BPW,), jnp.int32),
            pltpu.VMEM((_BPW, EMB_DIM), jnp.float32),
            pltpu.SemaphoreType.DMA,
        ],
    )


def _tc_fuse_body(s_hbm, t_ref, phylo_ref, k_ref, p_ref, c_ref, o_ref,
                  w_ref, b_ref, out_hbm,
                  in_bufs, out_bufs, in_sems, out_sems):
    w = w_ref[...]
    ws = w[:, 0:EMB_DIM]
    wp = w[:, EMB_DIM:EMB_DIM + PHYLO_DIM]
    wt = w[:, EMB_DIM + PHYLO_DIM:FUSED_IN]
    dn = (((1,), (1,)), ((), ()))
    tax = jnp.concatenate([k_ref[0:1, :], p_ref[0:1, :],
                           c_ref[0:1, :], o_ref[0:1, :]], axis=1)
    c = lax.dot_general(tax, wt, dn,
                        preferred_element_type=jnp.float32) \
        + jnp.reshape(b_ref[...], (1, EMB_DIM))
    p = lax.dot_general(phylo_ref[...], wp, dn,
                        preferred_element_type=jnp.float32)

    def in_copy(ci):
        slot = ci % _NBUF
        return pltpu.make_async_copy(
            s_hbm.at[pl.ds(ci * _CHUNK, _CHUNK)], in_bufs.at[slot],
            in_sems.at[slot])

    def out_copy(ci):
        slot = ci % _NBUF
        return pltpu.make_async_copy(
            out_bufs.at[slot], out_hbm.at[pl.ds(ci * _CHUNK, _CHUNK)],
            out_sems.at[slot])

    for ci in range(_NBUF):
        in_copy(ci).start()
    for ci in range(_NCHUNK):
        slot = ci % _NBUF
        in_copy(ci).wait()
        t_row = jnp.reshape(t_ref[pl.ds(ci * _CHUNK, _CHUNK)], (1, _CHUNK))
        oht = (t_row == lax.broadcasted_iota(
            jnp.int32, (NUM_PHYLO, _CHUNK), 0)).astype(jnp.float32)
        acc = lax.dot_general(in_bufs[slot], ws, dn,
                              preferred_element_type=jnp.float32)
        acc += lax.dot_general(oht, p, (((0,), (0,)), ((), ())),
                               preferred_element_type=jnp.float32)
        if ci >= _NBUF:
            out_copy(ci - _NBUF).wait()
        out_bufs[slot] = acc + c
        out_copy(ci).start()
        if ci + _NBUF < _NCHUNK:
            in_copy(ci + _NBUF).start()
    for ci in range(_NCHUNK - _NBUF, _NCHUNK):
        out_copy(ci).wait()


_tc_fuse = pl.pallas_call(
    _tc_fuse_body,
    in_specs=[
        pl.BlockSpec(memory_space=pl.ANY),
        pl.BlockSpec((B,), lambda: (0,)),
        pl.BlockSpec((NUM_PHYLO, PHYLO_DIM), lambda: (0, 0)),
        pl.BlockSpec((10, 32), lambda: (0, 0)),
        pl.BlockSpec((20, 32), lambda: (0, 0)),
        pl.BlockSpec((30, 32), lambda: (0, 0)),
        pl.BlockSpec((50, 32), lambda: (0, 0)),
        pl.BlockSpec((EMB_DIM, FUSED_IN), lambda: (0, 0)),
        pl.BlockSpec((EMB_DIM,), lambda: (0,)),
    ],
    out_specs=pl.BlockSpec(memory_space=pl.ANY),
    out_shape=jax.ShapeDtypeStruct((B, EMB_DIM), jnp.float32),
    scratch_shapes=[
        pltpu.VMEM((_NBUF, _CHUNK, EMB_DIM), jnp.float32),
        pltpu.VMEM((_NBUF, _CHUNK, EMB_DIM), jnp.float32),
        pltpu.SemaphoreType.DMA((_NBUF,)),
        pltpu.SemaphoreType.DMA((_NBUF,)),
    ],
)


def kernel(species_ids, divergence_times, species_table, phylo_table,
           kingdom_table, phylum_table, class_table, order_table, W, b):
    ids = species_ids.astype(jnp.int32)
    s_rows = _get_sc_gather()(species_table, ids)
    times = divergence_times.astype(jnp.int32)
    return _tc_fuse(s_rows, times, phylo_table, kingdom_table,
                    phylum_table, class_table, order_table, W, b)


# speedup vs baseline: 1.0030x; 1.0030x over previous
import functools

import jax
import jax.numpy as jnp
from jax import lax
from jax.experimental import pallas as pl
from jax.experimental.pallas import tpu as pltpu
from jax.experimental.pallas import tpu_sc as plsc

B = 16384
EMB_DIM = 128
PHYLO_DIM = 64
FUSED_IN = 320
NUM_PHYLO = 100

_NC = 2
_NS = 16
_NW = _NC * _NS
_BPW = B // _NW
_CH = 128

_CHUNK = 1024
_NCHUNK = B // _CHUNK
_NBUF = 6


def _sc_gather_body(species_hbm, sid_hbm, s_out, sidx_v, srows_v, sem):
    wid = lax.axis_index("s") * _NC + lax.axis_index("c")
    base = wid * _BPW
    pltpu.sync_copy(sid_hbm.at[pl.ds(base, _BPW)], sidx_v)
    gathers = []
    for j in range(_BPW // _CH):
        gathers.append(pltpu.async_copy(
            species_hbm.at[sidx_v.at[pl.ds(j * _CH, _CH)]],
            srows_v.at[pl.ds(j * _CH, _CH)], sem))
    for g in gathers:
        g.wait()
    pltpu.sync_copy(srows_v, s_out.at[pl.ds(base, _BPW)])


@functools.lru_cache(maxsize=None)
def _get_sc_gather():
    return pl.kernel(
        _sc_gather_body,
        out_type=jax.ShapeDtypeStruct((B, EMB_DIM), jnp.float32),
        mesh=plsc.VectorSubcoreMesh(core_axis_name="c", subcore_axis_name="s"),
        scratch_types=[
            pltpu.VMEM((_BPW,), jnp.int32),
            pltpu.VMEM((_BPW, EMB_DIM), jnp.float32),
            pltpu.SemaphoreType.DMA,
        ],
    )


def _tc_fuse_body(s_hbm, t_ref, phylo_ref, k_ref, p_ref, c_ref, o_ref,
                  w_ref, b_ref, out_hbm,
                  in_bufs, out_bufs, in_sems, out_sems):
    w = w_ref[...]
    ws = w[:, 0:EMB_DIM]
    wp = w[:, EMB_DIM:EMB_DIM + PHYLO_DIM]
    wt = w[:, EMB_DIM + PHYLO_DIM:FUSED_IN]
    dn = (((1,), (1,)), ((), ()))
    tax = jnp.concatenate([k_ref[0:1, :], p_ref[0:1, :],
                           c_ref[0:1, :], o_ref[0:1, :]], axis=1)
    c = lax.dot_general(tax, wt, dn,
                        preferred_element_type=jnp.float32) \
        + jnp.reshape(b_ref[...], (1, EMB_DIM))
    p = lax.dot_general(phylo_ref[...], wp, dn,
                        preferred_element_type=jnp.float32)

    def in_copy(ci):
        slot = ci % _NBUF
        return pltpu.make_async_copy(
            s_hbm.at[pl.ds(ci * _CHUNK, _CHUNK)], in_bufs.at[slot],
            in_sems.at[slot])

    def out_copy(ci):
        slot = ci % _NBUF
        return pltpu.make_async_copy(
            out_bufs.at[slot], out_hbm.at[pl.ds(ci * _CHUNK, _CHUNK)],
            out_sems.at[slot])

    for ci in range(_NBUF):
        in_copy(ci).start()
    for ci in range(_NCHUNK):
        slot = ci % _NBUF
        in_copy(ci).wait()
        t_row = jnp.reshape(t_ref[pl.ds(ci * _CHUNK, _CHUNK)], (1, _CHUNK))
        oht = (t_row == lax.broadcasted_iota(
            jnp.int32, (NUM_PHYLO, _CHUNK), 0)).astype(jnp.bfloat16)
        acc = lax.dot_general(in_bufs[slot].astype(jnp.bfloat16),
                              ws.astype(jnp.bfloat16), dn,
                              preferred_element_type=jnp.float32)
        acc += lax.dot_general(oht, p.astype(jnp.bfloat16),
                               (((0,), (0,)), ((), ())),
                               preferred_element_type=jnp.float32)
        if ci >= _NBUF:
            out_copy(ci - _NBUF).wait()
        out_bufs[slot] = acc + c
        out_copy(ci).start()
        if ci + _NBUF < _NCHUNK:
            in_copy(ci + _NBUF).start()
    for ci in range(_NCHUNK - _NBUF, _NCHUNK):
        out_copy(ci).wait()


_tc_fuse = pl.pallas_call(
    _tc_fuse_body,
    in_specs=[
        pl.BlockSpec(memory_space=pl.ANY),
        pl.BlockSpec((B,), lambda: (0,)),
        pl.BlockSpec((NUM_PHYLO, PHYLO_DIM), lambda: (0, 0)),
        pl.BlockSpec((10, 32), lambda: (0, 0)),
        pl.BlockSpec((20, 32), lambda: (0, 0)),
        pl.BlockSpec((30, 32), lambda: (0, 0)),
        pl.BlockSpec((50, 32), lambda: (0, 0)),
        pl.BlockSpec((EMB_DIM, FUSED_IN), lambda: (0, 0)),
        pl.BlockSpec((EMB_DIM,), lambda: (0,)),
    ],
    out_specs=pl.BlockSpec(memory_space=pl.ANY),
    out_shape=jax.ShapeDtypeStruct((B, EMB_DIM), jnp.float32),
    scratch_shapes=[
        pltpu.VMEM((_NBUF, _CHUNK, EMB_DIM), jnp.float32),
        pltpu.VMEM((_NBUF, _CHUNK, EMB_DIM), jnp.float32),
        pltpu.SemaphoreType.DMA((_NBUF,)),
        pltpu.SemaphoreType.DMA((_NBUF,)),
    ],
)


def kernel(species_ids, divergence_times, species_table, phylo_table,
           kingdom_table, phylum_table, class_table, order_table, W, b):
    ids = species_ids.astype(jnp.int32)
    s_rows = _get_sc_gather()(species_table, ids)
    times = divergence_times.astype(jnp.int32)
    return _tc_fuse(s_rows, times, phylo_table, kingdom_table,
                    phylum_table, class_table, order_table, W, b)
